# baseline (speedup 1.0000x reference)
import functools

import jax
import jax.numpy as jnp
from jax import lax
from jax.experimental import pallas as pl
from jax.experimental.pallas import tpu as pltpu
from jax.experimental.pallas import tpu_sc as plsc


def _make_gather(V, D, B):
    info = plsc.get_sparse_core_info()
    NC, NS = info.num_cores, info.num_subcores
    NW = NC * NS
    assert B % NW == 0
    b_per_w = B // NW
    CHUNK = 512
    assert b_per_w % CHUNK == 0
    n_chunks = b_per_w // CHUNK

    mesh = plsc.VectorSubcoreMesh(core_axis_name="c", subcore_axis_name="s")

    @functools.partial(
        pl.kernel,
        out_type=jax.ShapeDtypeStruct((B, D), jnp.float32),
        mesh=mesh,
        scratch_types=[
            pltpu.VMEM((CHUNK,), jnp.int32),
            pltpu.VMEM((CHUNK, D), jnp.float32),
            pltpu.SemaphoreType.DMA,
        ],
        compiler_params=pltpu.CompilerParams(use_tc_tiling_on_sc=False),
    )
    def gather_kernel(idx_hbm, table_hbm, out_hbm, idx_v, rows_v, sem):
        wid = lax.axis_index("s") * NC + lax.axis_index("c")
        base = wid * b_per_w

        def body(c, carry):
            off = base + c * CHUNK
            pltpu.sync_copy(idx_hbm.at[pl.ds(off, CHUNK)], idx_v)
            pltpu.async_copy(table_hbm.at[idx_v], rows_v, sem).wait()
            pltpu.sync_copy(rows_v, out_hbm.at[pl.ds(off, CHUNK)])
            return carry

        lax.fori_loop(0, n_chunks, body, 0)

    return gather_kernel


def kernel(x, weight):
    V, D = weight.shape
    B = x.size
    xf = x.reshape(B).astype(jnp.int32)
    out = _make_gather(V, D, B)(xf, weight)
    return out.reshape(x.shape + (D,))


# speedup vs baseline: 1.0422x; 1.0422x over previous
import functools

import jax
import jax.numpy as jnp
from jax import lax
from jax.experimental import pallas as pl
from jax.experimental.pallas import tpu as pltpu
from jax.experimental.pallas import tpu_sc as plsc

_NBUF = 4
_CHUNK = 400


def _make_gather(V, D, B):
    info = plsc.get_sparse_core_info()
    NC, NS = info.num_cores, info.num_subcores
    NW = NC * NS
    assert B % NW == 0
    b_per_w = B // NW
    chunk = _CHUNK
    assert b_per_w % chunk == 0
    n_chunks = b_per_w // chunk
    assert n_chunks >= _NBUF and (n_chunks - 4) % _NBUF == 0

    mesh = plsc.VectorSubcoreMesh(core_axis_name="c", subcore_axis_name="s")

    scratch = [pltpu.VMEM((b_per_w,), jnp.int32)]
    scratch += [pltpu.VMEM((chunk, D), jnp.float32) for _ in range(_NBUF)]
    scratch += [pltpu.SemaphoreType.DMA for _ in range(2 * _NBUF)]

    @functools.partial(
        pl.kernel,
        out_type=jax.ShapeDtypeStruct((B, D), jnp.float32),
        mesh=mesh,
        scratch_types=scratch,
        compiler_params=pltpu.CompilerParams(use_tc_tiling_on_sc=False),
    )
    def gather_kernel(idx_hbm, table_hbm, out_hbm, idx_all, *bufs):
        rows = bufs[:_NBUF]
        gsem = bufs[_NBUF:2 * _NBUF]
        wsem = bufs[2 * _NBUF:]
        wid = lax.axis_index("s") * NC + lax.axis_index("c")
        base = wid * b_per_w

        pltpu.sync_copy(idx_hbm.at[pl.ds(base, b_per_w)], idx_all)

        def g_copy(c, b):
            return pltpu.make_async_copy(
                table_hbm.at[idx_all.at[pl.ds(c * chunk, chunk)]],
                rows[b], gsem[b])

        def w_copy(c, b):
            return pltpu.make_async_copy(
                rows[b], out_hbm.at[pl.ds(base + c * chunk, chunk)], wsem[b])

        g_copy(0, 0).start()
        g_copy(1, 1).start()
        g_copy(0, 0).wait()
        w_copy(0, 0).start()
        g_copy(2, 2).start()
        g_copy(1, 1).wait()
        w_copy(1, 1).start()
        g_copy(3, 3).start()

        def body(p, carry):
            c0 = 2 + p * _NBUF
            for j in range(_NBUF):
                c = c0 + j
                b = (2 + j) % _NBUF
                b2 = j
                g_copy(c, b).wait()
                w_copy(c, b).start()
                w_copy(c - 2, b2).wait()
                g_copy(c + 2, b2).start()
            return carry

        n_main = (n_chunks - 4) // _NBUF
        lax.fori_loop(0, n_main, body, 0)

        cA, cB = n_chunks - 2, n_chunks - 1
        bA, bB = cA % _NBUF, cB % _NBUF
        g_copy(cA, bA).wait()
        w_copy(cA, bA).start()
        g_copy(cB, bB).wait()
        w_copy(cB, bB).start()
        w_copy(n_chunks - 4, (n_chunks - 4) % _NBUF).wait()
        w_copy(n_chunks - 3, (n_chunks - 3) % _NBUF).wait()
        w_copy(cA, bA).wait()
        w_copy(cB, bB).wait()

    return gather_kernel


def kernel(x, weight):
    V, D = weight.shape
    B = x.size
    xf = x.reshape(B).astype(jnp.int32)
    out = _make_gather(V, D, B)(xf, weight)
    return out.reshape(x.shape + (D,))
